# baseline (speedup 1.0000x reference)
import jax
import jax.numpy as jnp
from jax.experimental import pallas as pl


def _gather_kernel(x_ref, o_ref):
    a = x_ref[1, 2]
    b = x_ref[2, 1]
    col = jax.lax.broadcasted_iota(jnp.int32, (8, 128), 1)
    o_ref[...] = jnp.where(col == 0, a, jnp.where(col == 1, b, 0.0))


def kernel(x):
    out = pl.pallas_call(
        _gather_kernel,
        grid=(1,),
        in_specs=[pl.BlockSpec((8, 128), lambda i: (0, 0))],
        out_specs=pl.BlockSpec((8, 128), lambda i: (0, 0)),
        out_shape=jax.ShapeDtypeStruct((8, 128), jnp.float32),
    )(x)
    return out[0, :2]


# speedup vs baseline: 1.9215x; 1.9215x over previous
import jax
import jax.numpy as jnp
from jax.experimental import pallas as pl


def _gather_kernel(x_ref, o_ref):
    a = x_ref[1, 2]
    b = x_ref[2, 1]
    col = jax.lax.iota(jnp.int32, 2)
    o_ref[...] = jnp.where(col == 0, a, b)


def kernel(x):
    return pl.pallas_call(
        _gather_kernel,
        grid=(1,),
        in_specs=[pl.BlockSpec((8, 128), lambda i: (0, 0))],
        out_specs=pl.BlockSpec((2,), lambda i: (0,)),
        out_shape=jax.ShapeDtypeStruct((2,), jnp.float32),
    )(x)
